# baseline (speedup 1.0000x reference)
import functools

import jax
import jax.numpy as jnp
from jax import lax
from jax.experimental import pallas as pl
from jax.experimental.pallas import tpu as pltpu
from jax.experimental.pallas import tpu_sc as plsc

N = 8192
K = 16
B = 8
RB = 256
CB = 256
NR = N // RB
NC = N // CB
TB = 512
NP = 8
PF = 128
NF = 32
INF = float("inf")


def _knn_body(sp_ref, pos_all_ref, posT_r_ref, batch_all_ref, batchT_r_ref,
              out_ref, v_ref, i_ref):
    r = pl.program_id(0)
    jlo = sp_ref[0, r]
    jhi = sp_ref[1, r]
    xr = posT_r_ref[...]
    btr = batchT_r_ref[...]

    v_ref[...] = jnp.full((K, RB), INF, jnp.float32)
    i_ref[...] = jnp.zeros((K, RB), jnp.float32)

    def _merge(c, _):
        xc = pos_all_ref[pl.ds(c * CB, CB), :]
        d0 = ((xc[:, 0:1] - xr[0:1, :]) ** 2
              + (xc[:, 1:2] - xr[1:2, :]) ** 2
              + (xc[:, 2:3] - xr[2:3, :]) ** 2)
        cross = batch_all_ref[pl.ds(c * CB, CB), :] != btr
        d0 = jnp.where(cross, INF, d0)
        cidx = ((c * CB).astype(jnp.float32)
                + lax.broadcasted_iota(jnp.int32, (CB, RB), 0)
                  .astype(jnp.float32))

        vold = v_ref[...]
        iold = i_ref[...]
        vmax = jnp.max(vold, axis=0)
        do_merge = jnp.any(d0 < vmax[None, :])

        @pl.when(do_merge)
        def _loop():
            d = d0
            vo = vold
            io = iold
            for t in range(K):
                m = jnp.minimum(jnp.min(d, axis=0), jnp.min(vo, axis=0))
                mr = m[None, :]
                sel_d = jnp.min(jnp.where(d == mr, cidx, INF), axis=0)
                sel_v = jnp.min(jnp.where(vo == mr, io, INF), axis=0)
                sel = jnp.minimum(sel_d, sel_v)
                v_ref[t, :] = m
                i_ref[t, :] = sel
                sr = sel[None, :]
                d = jnp.where(cidx == sr, INF, d)
                vo = jnp.where(io == sr, INF, vo)

        return 0

    lax.fori_loop(jlo, jhi + 1, _merge, 0)
    out_ref[...] = i_ref[...].astype(jnp.int32)


def _knn_idx(sp, pos, posT, batch2d, batchT, ro):
    nrh = NR // NP
    grid_spec = pltpu.PrefetchScalarGridSpec(
        num_scalar_prefetch=1,
        grid=(nrh,),
        in_specs=[
            pl.BlockSpec((N, 3), lambda r, sp: (0, 0)),
            pl.BlockSpec((3, RB), lambda r, sp: (0, ro * nrh + r)),
            pl.BlockSpec((N, 1), lambda r, sp: (0, 0)),
            pl.BlockSpec((1, RB), lambda r, sp: (0, ro * nrh + r)),
        ],
        out_specs=pl.BlockSpec((K, RB), lambda r, sp: (0, r)),
        scratch_shapes=[
            pltpu.VMEM((K, RB), jnp.float32),
            pltpu.VMEM((K, RB), jnp.float32),
        ],
    )
    return pl.pallas_call(
        _knn_body,
        grid_spec=grid_spec,
        out_shape=jax.ShapeDtypeStruct((K, N // NP), jnp.int32),
        compiler_params=pltpu.CompilerParams(
            dimension_semantics=("parallel",)),
    )(sp, pos, posT, batch2d, batchT)


def _sc_gather(table, idx_flat, d):
    e = idx_flat.shape[0]
    n_workers = 32
    per_w = e // n_workers
    ch = 256
    n_ch = per_w // ch
    mesh = plsc.VectorSubcoreMesh(core_axis_name="c", subcore_axis_name="s")

    @functools.partial(
        pl.kernel,
        mesh=mesh,
        out_type=jax.ShapeDtypeStruct((e, d), jnp.float32),
        scratch_types=[
            pltpu.VMEM((per_w,), jnp.int32),
            pltpu.VMEM((ch, d), jnp.float32),
            pltpu.VMEM((ch, d), jnp.float32),
            pltpu.SemaphoreType.DMA,
            pltpu.SemaphoreType.DMA,
        ],
    )
    def _gather_kernel(table_hbm, idx_hbm, out_hbm, idx_v, buf_a, buf_b,
                       sem_a, sem_b):
        wid = lax.axis_index("s") * 2 + lax.axis_index("c")
        base = wid * per_w
        pltpu.sync_copy(idx_hbm.at[pl.ds(base, per_w)], idx_v)
        bufs = (buf_a, buf_b)
        sems = (sem_a, sem_b)
        copies = [pltpu.async_copy(table_hbm.at[idx_v.at[pl.ds(0, ch)]],
                                   buf_a, sem_a)]
        for j in range(n_ch):
            if j + 1 < n_ch:
                copies.append(pltpu.async_copy(
                    table_hbm.at[idx_v.at[pl.ds((j + 1) * ch, ch)]],
                    bufs[(j + 1) % 2], sems[(j + 1) % 2]))
            copies[j].wait()
            pltpu.sync_copy(bufs[j % 2], out_hbm.at[pl.ds(base + j * ch, ch)])

    return _gather_kernel(table, idx_flat)


def _layer1_body(pj_ref, pos_ref, wsum_ref, wbot_ref, b1a_ref, w1b_ref,
                 b1b_ref, out_ref):
    posbot = jnp.dot(pos_ref[...], wbot_ref[...],
                     preferred_element_type=jnp.float32)
    pj_all = pj_ref[...].reshape(K * TB, PF)
    pre = (jnp.dot(pj_all, wsum_ref[...],
                   preferred_element_type=jnp.float32)
           .reshape(K, TB, 32) - posbot[None] + b1a_ref[...][None])
    msg = (jnp.dot(jnp.maximum(pre, 0.0).reshape(K * TB, 32), w1b_ref[...],
                   preferred_element_type=jnp.float32)
           .reshape(K, TB, 32) + b1b_ref[...][None])
    out_ref[...] = jnp.maximum(jnp.max(msg, axis=0), 0.0)


def _layer2_body(hj_ref, pj_ref, pos_ref, batch_ref, w2ah_ref, w2ap_ref,
                 b2a_ref, w2b_ref, b2b_ref, out_ref, g_ref):
    t = pl.program_id(0)
    nt = pl.num_programs(0)
    posbot = jnp.dot(pos_ref[...], w2ap_ref[...],
                     preferred_element_type=jnp.float32)
    hj_all = hj_ref[...].reshape(K * TB, PF)
    pj_all = pj_ref[...].reshape(K * TB, PF)
    pre = ((jnp.dot(hj_all, w2ah_ref[...],
                    preferred_element_type=jnp.float32)
            + jnp.dot(pj_all, w2ap_ref[...],
                      preferred_element_type=jnp.float32))
           .reshape(K, TB, 32) - posbot[None] + b2a_ref[...][None])
    msg = (jnp.dot(jnp.maximum(pre, 0.0).reshape(K * TB, 32), w2b_ref[...],
                   preferred_element_type=jnp.float32)
           .reshape(K, TB, 32) + b2b_ref[...][None])
    h2 = jnp.maximum(jnp.max(msg, axis=0), 0.0)

    @pl.when(t == 0)
    def _init():
        g_ref[...] = jnp.full((B, 32), -INF, jnp.float32)

    bcol = batch_ref[...]
    for b in range(B):
        seg = jnp.max(jnp.where(bcol == b, h2, -INF), axis=0, keepdims=True)
        g_ref[b:b + 1, :] = jnp.maximum(g_ref[b:b + 1, :], seg)

    @pl.when(t == nt - 1)
    def _emit():
        out_ref[...] = g_ref[...]


def _run_layer1(pj, pos_pad, wsum, wbot, b1a2, w1b, b1b2, ro):
    nth = N // NP // TB
    return pl.pallas_call(
        _layer1_body,
        grid=(nth,),
        in_specs=[
            pl.BlockSpec((K, TB, PF), lambda t: (0, t, 0)),
            pl.BlockSpec((TB, PF), lambda t: (ro * nth + t, 0)),
            pl.BlockSpec((PF, 32), lambda t: (0, 0)),
            pl.BlockSpec((PF, 32), lambda t: (0, 0)),
            pl.BlockSpec((1, 32), lambda t: (0, 0)),
            pl.BlockSpec((32, 32), lambda t: (0, 0)),
            pl.BlockSpec((1, 32), lambda t: (0, 0)),
        ],
        out_specs=pl.BlockSpec((TB, 32), lambda t: (t, 0)),
        out_shape=jax.ShapeDtypeStruct((N // NP, 32), jnp.float32),
        compiler_params=pltpu.CompilerParams(
            dimension_semantics=("parallel",)),
    )(pj, pos_pad, wsum, wbot, b1a2, w1b, b1b2)


def _run_layer2(hj, pj, pos_pad, batch2d, w2ah, w2ap, b2a2, w2b, b2b2, ro):
    nth = N // NP // TB
    return pl.pallas_call(
        _layer2_body,
        grid=(nth,),
        in_specs=[
            pl.BlockSpec((K, TB, PF), lambda t: (0, t, 0)),
            pl.BlockSpec((K, TB, PF), lambda t: (0, t, 0)),
            pl.BlockSpec((TB, PF), lambda t: (ro * nth + t, 0)),
            pl.BlockSpec((TB, 1), lambda t: (ro * nth + t, 0)),
            pl.BlockSpec((PF, 32), lambda t: (0, 0)),
            pl.BlockSpec((PF, 32), lambda t: (0, 0)),
            pl.BlockSpec((1, 32), lambda t: (0, 0)),
            pl.BlockSpec((32, 32), lambda t: (0, 0)),
            pl.BlockSpec((1, 32), lambda t: (0, 0)),
        ],
        out_specs=pl.BlockSpec((B, 32), lambda t: (0, 0)),
        out_shape=jax.ShapeDtypeStruct((B, 32), jnp.float32),
        scratch_shapes=[pltpu.VMEM((B, 32), jnp.float32)],
    )(hj, pj, pos_pad, batch2d, w2ah, w2ap, b2a2, w2b, b2b2)


def _combine_body(*refs):
    g_refs = refs[:NP]
    wc_ref, bc_ref, out_ref = refs[NP], refs[NP + 1], refs[NP + 2]
    g = g_refs[0][...]
    for gr in g_refs[1:]:
        g = jnp.maximum(g, gr[...])
    out_ref[...] = (jnp.dot(g, wc_ref[...],
                            preferred_element_type=jnp.float32) + bc_ref[...])


def _combine(gs, wc, bc2):
    return pl.pallas_call(
        _combine_body,
        grid=(1,),
        in_specs=(
            [pl.BlockSpec((B, 32), lambda i: (0, 0)) for _ in range(NP)]
            + [pl.BlockSpec((32, 40), lambda i: (0, 0)),
               pl.BlockSpec((1, 40), lambda i: (0, 0))]),
        out_specs=pl.BlockSpec((B, 40), lambda i: (0, 0)),
        out_shape=jax.ShapeDtypeStruct((B, 40), jnp.float32),
    )(*gs, wc, bc2)


def kernel(pos, batch, W1a, b1a, W1b, b1b, W2a, b2a, W2b, b2b, Wc, bc):
    batch = batch.astype(jnp.int32)
    pos = pos.astype(jnp.float32)

    pos_pad = jnp.pad(pos, ((0, 0), (0, PF - 3)))
    posT = pos.T
    batch2d = batch.reshape(N, 1)
    batchT = batch.reshape(1, N)

    starts = jnp.searchsorted(batch, jnp.arange(B, dtype=jnp.int32),
                              side="left")
    ends = jnp.searchsorted(batch, jnp.arange(B, dtype=jnp.int32),
                            side="right")
    brows = batch.reshape(NR, RB)
    jlo = (starts[brows[:, 0]] // CB).astype(jnp.int32)
    jhi = ((ends[brows[:, -1]] - 1) // CB).astype(jnp.int32)
    sp = jnp.stack([jlo, jhi]).astype(jnp.int32)

    nrp = NR // NP
    flats = []
    for p in range(NP):
        idx_p = _knn_idx(sp[:, p * nrp:(p + 1) * nrp], pos, posT,
                         batch2d, batchT, p)
        flats.append(idx_p.reshape(-1))

    w1_top = jnp.pad(W1a[0:3], ((0, PF - 3), (0, 0)))
    w1_bot = jnp.pad(W1a[3:6], ((0, PF - 3), (0, 0)))
    w1_sum = w1_top + w1_bot
    w2ah = jnp.pad(W2a[0:32], ((0, PF - 32), (0, 0)))
    w2ap = jnp.pad(W2a[32:35], ((0, PF - 3), (0, 0)))
    b1a2 = b1a.reshape(1, 32)
    b1b2 = b1b.reshape(1, 32)
    b2a2 = b2a.reshape(1, 32)
    b2b2 = b2b.reshape(1, 32)
    bc2 = bc.reshape(1, 40)

    hn = N // NP

    pjs = [_sc_gather(pos_pad, f, PF).reshape(K, hn, PF) for f in flats]

    h1s = [_run_layer1(pjs[p], pos_pad, w1_sum, w1_bot, b1a2, W1b, b1b2, p)
           for p in range(NP)]
    h1p = jnp.pad(jnp.concatenate(h1s, axis=0), ((0, 0), (0, PF - 32)))

    hjs = [_sc_gather(h1p, f, PF).reshape(K, hn, PF) for f in flats]

    gs = [_run_layer2(hjs[p], pjs[p], pos_pad, batch2d, w2ah, w2ap, b2a2,
                      W2b, b2b2, p) for p in range(NP)]

    return _combine(gs, Wc, bc2)


# speedup vs baseline: 1.1202x; 1.1202x over previous
import functools

import jax
import jax.numpy as jnp
from jax import lax
from jax.experimental import pallas as pl
from jax.experimental.pallas import tpu as pltpu
from jax.experimental.pallas import tpu_sc as plsc

N = 8192
K = 16
B = 8
RB = 256
CB = 256
NR = N // RB
NC = N // CB
TB = 512
NP = 4
PF = 128
NF = 32
INF = float("inf")


def _knn_body(sp_ref, pos_all_ref, posT_r_ref, batch_all_ref, batchT_r_ref,
              out_ref, v_ref, i_ref):
    r = pl.program_id(0)
    jlo = sp_ref[0, r]
    jhi = sp_ref[1, r]
    xr = posT_r_ref[...]
    btr = batchT_r_ref[...]

    v_ref[...] = jnp.full((K, RB), INF, jnp.float32)
    i_ref[...] = jnp.zeros((K, RB), jnp.float32)

    def _merge(c, _):
        xc = pos_all_ref[pl.ds(c * CB, CB), :]
        d0 = ((xc[:, 0:1] - xr[0:1, :]) ** 2
              + (xc[:, 1:2] - xr[1:2, :]) ** 2
              + (xc[:, 2:3] - xr[2:3, :]) ** 2)
        cross = batch_all_ref[pl.ds(c * CB, CB), :] != btr
        d0 = jnp.where(cross, INF, d0)
        cidx = ((c * CB).astype(jnp.float32)
                + lax.broadcasted_iota(jnp.int32, (CB, RB), 0)
                  .astype(jnp.float32))

        vold = v_ref[...]
        iold = i_ref[...]
        vmax = jnp.max(vold, axis=0)
        do_merge = jnp.any(d0 < vmax[None, :])

        @pl.when(do_merge)
        def _loop():
            d = d0
            vo = vold
            io = iold
            for t in range(K):
                m = jnp.minimum(jnp.min(d, axis=0), jnp.min(vo, axis=0))
                mr = m[None, :]
                sel_d = jnp.min(jnp.where(d == mr, cidx, INF), axis=0)
                sel_v = jnp.min(jnp.where(vo == mr, io, INF), axis=0)
                sel = jnp.minimum(sel_d, sel_v)
                v_ref[t, :] = m
                i_ref[t, :] = sel
                sr = sel[None, :]
                d = jnp.where(cidx == sr, INF, d)
                vo = jnp.where(io == sr, INF, vo)

        return 0

    lax.fori_loop(jlo, jhi + 1, _merge, 0)
    out_ref[...] = i_ref[...].astype(jnp.int32)


def _knn_idx(sp, pos, posT, batch2d, batchT, ro):
    nrh = NR // NP
    grid_spec = pltpu.PrefetchScalarGridSpec(
        num_scalar_prefetch=1,
        grid=(nrh,),
        in_specs=[
            pl.BlockSpec((N, 3), lambda r, sp: (0, 0)),
            pl.BlockSpec((3, RB), lambda r, sp: (0, ro * nrh + r)),
            pl.BlockSpec((N, 1), lambda r, sp: (0, 0)),
            pl.BlockSpec((1, RB), lambda r, sp: (0, ro * nrh + r)),
        ],
        out_specs=pl.BlockSpec((K, RB), lambda r, sp: (0, r)),
        scratch_shapes=[
            pltpu.VMEM((K, RB), jnp.float32),
            pltpu.VMEM((K, RB), jnp.float32),
        ],
    )
    return pl.pallas_call(
        _knn_body,
        grid_spec=grid_spec,
        out_shape=jax.ShapeDtypeStruct((K, N // NP), jnp.int32),
        compiler_params=pltpu.CompilerParams(
            dimension_semantics=("parallel",)),
    )(sp, pos, posT, batch2d, batchT)


def _sc_gather(table, idx_flat, d):
    e = idx_flat.shape[0]
    n_workers = 32
    per_w = e // n_workers
    ch = 256
    n_ch = per_w // ch
    mesh = plsc.VectorSubcoreMesh(core_axis_name="c", subcore_axis_name="s")

    @functools.partial(
        pl.kernel,
        mesh=mesh,
        out_type=jax.ShapeDtypeStruct((e, d), jnp.float32),
        scratch_types=[
            pltpu.VMEM((per_w,), jnp.int32),
            pltpu.VMEM((ch, d), jnp.float32),
            pltpu.VMEM((ch, d), jnp.float32),
            pltpu.SemaphoreType.DMA,
            pltpu.SemaphoreType.DMA,
        ],
    )
    def _gather_kernel(table_hbm, idx_hbm, out_hbm, idx_v, buf_a, buf_b,
                       sem_a, sem_b):
        wid = lax.axis_index("s") * 2 + lax.axis_index("c")
        base = wid * per_w
        pltpu.sync_copy(idx_hbm.at[pl.ds(base, per_w)], idx_v)
        bufs = (buf_a, buf_b)
        sems = (sem_a, sem_b)
        copies = [pltpu.async_copy(table_hbm.at[idx_v.at[pl.ds(0, ch)]],
                                   buf_a, sem_a)]
        for j in range(n_ch):
            if j + 1 < n_ch:
                copies.append(pltpu.async_copy(
                    table_hbm.at[idx_v.at[pl.ds((j + 1) * ch, ch)]],
                    bufs[(j + 1) % 2], sems[(j + 1) % 2]))
            copies[j].wait()
            pltpu.sync_copy(bufs[j % 2], out_hbm.at[pl.ds(base + j * ch, ch)])

    return _gather_kernel(table, idx_flat)


def _layer1_body(pj_ref, pos_ref, wsum_ref, wbot_ref, b1a_ref, w1b_ref,
                 b1b_ref, out_ref):
    posbot = jnp.dot(pos_ref[...], wbot_ref[...],
                     preferred_element_type=jnp.float32)
    pj_all = pj_ref[...].reshape(K * TB, PF)
    pre = (jnp.dot(pj_all, wsum_ref[...],
                   preferred_element_type=jnp.float32)
           .reshape(K, TB, 32) - posbot[None] + b1a_ref[...][None])
    msg = (jnp.dot(jnp.maximum(pre, 0.0).reshape(K * TB, 32), w1b_ref[...],
                   preferred_element_type=jnp.float32)
           .reshape(K, TB, 32) + b1b_ref[...][None])
    out_ref[...] = jnp.maximum(jnp.max(msg, axis=0), 0.0)


def _layer2_body(hj_ref, pj_ref, pos_ref, batch_ref, w2ah_ref, w2ap_ref,
                 b2a_ref, w2b_ref, b2b_ref, out_ref, g_ref):
    t = pl.program_id(0)
    nt = pl.num_programs(0)
    posbot = jnp.dot(pos_ref[...], w2ap_ref[...],
                     preferred_element_type=jnp.float32)
    hj_all = hj_ref[...].reshape(K * TB, PF)
    pj_all = pj_ref[...].reshape(K * TB, PF)
    pre = ((jnp.dot(hj_all, w2ah_ref[...],
                    preferred_element_type=jnp.float32)
            + jnp.dot(pj_all, w2ap_ref[...],
                      preferred_element_type=jnp.float32))
           .reshape(K, TB, 32) - posbot[None] + b2a_ref[...][None])
    msg = (jnp.dot(jnp.maximum(pre, 0.0).reshape(K * TB, 32), w2b_ref[...],
                   preferred_element_type=jnp.float32)
           .reshape(K, TB, 32) + b2b_ref[...][None])
    h2 = jnp.maximum(jnp.max(msg, axis=0), 0.0)

    @pl.when(t == 0)
    def _init():
        g_ref[...] = jnp.full((B, 32), -INF, jnp.float32)

    bcol = batch_ref[...]
    for b in range(B):
        seg = jnp.max(jnp.where(bcol == b, h2, -INF), axis=0, keepdims=True)
        g_ref[b:b + 1, :] = jnp.maximum(g_ref[b:b + 1, :], seg)

    @pl.when(t == nt - 1)
    def _emit():
        out_ref[...] = g_ref[...]


def _run_layer1(pj, pos_pad, wsum, wbot, b1a2, w1b, b1b2, ro):
    nth = N // NP // TB
    return pl.pallas_call(
        _layer1_body,
        grid=(nth,),
        in_specs=[
            pl.BlockSpec((K, TB, PF), lambda t: (0, t, 0)),
            pl.BlockSpec((TB, PF), lambda t: (ro * nth + t, 0)),
            pl.BlockSpec((PF, 32), lambda t: (0, 0)),
            pl.BlockSpec((PF, 32), lambda t: (0, 0)),
            pl.BlockSpec((1, 32), lambda t: (0, 0)),
            pl.BlockSpec((32, 32), lambda t: (0, 0)),
            pl.BlockSpec((1, 32), lambda t: (0, 0)),
        ],
        out_specs=pl.BlockSpec((TB, 32), lambda t: (t, 0)),
        out_shape=jax.ShapeDtypeStruct((N // NP, 32), jnp.float32),
        compiler_params=pltpu.CompilerParams(
            dimension_semantics=("parallel",)),
    )(pj, pos_pad, wsum, wbot, b1a2, w1b, b1b2)


def _run_layer2(hj, pj, pos_pad, batch2d, w2ah, w2ap, b2a2, w2b, b2b2, ro):
    nth = N // NP // TB
    return pl.pallas_call(
        _layer2_body,
        grid=(nth,),
        in_specs=[
            pl.BlockSpec((K, TB, PF), lambda t: (0, t, 0)),
            pl.BlockSpec((K, TB, PF), lambda t: (0, t, 0)),
            pl.BlockSpec((TB, PF), lambda t: (ro * nth + t, 0)),
            pl.BlockSpec((TB, 1), lambda t: (ro * nth + t, 0)),
            pl.BlockSpec((PF, 32), lambda t: (0, 0)),
            pl.BlockSpec((PF, 32), lambda t: (0, 0)),
            pl.BlockSpec((1, 32), lambda t: (0, 0)),
            pl.BlockSpec((32, 32), lambda t: (0, 0)),
            pl.BlockSpec((1, 32), lambda t: (0, 0)),
        ],
        out_specs=pl.BlockSpec((B, 32), lambda t: (0, 0)),
        out_shape=jax.ShapeDtypeStruct((B, 32), jnp.float32),
        scratch_shapes=[pltpu.VMEM((B, 32), jnp.float32)],
    )(hj, pj, pos_pad, batch2d, w2ah, w2ap, b2a2, w2b, b2b2)


def _combine_body(*refs):
    g_refs = refs[:NP]
    wc_ref, bc_ref, out_ref = refs[NP], refs[NP + 1], refs[NP + 2]
    g = g_refs[0][...]
    for gr in g_refs[1:]:
        g = jnp.maximum(g, gr[...])
    out_ref[...] = (jnp.dot(g, wc_ref[...],
                            preferred_element_type=jnp.float32) + bc_ref[...])


def _combine(gs, wc, bc2):
    return pl.pallas_call(
        _combine_body,
        grid=(1,),
        in_specs=(
            [pl.BlockSpec((B, 32), lambda i: (0, 0)) for _ in range(NP)]
            + [pl.BlockSpec((32, 40), lambda i: (0, 0)),
               pl.BlockSpec((1, 40), lambda i: (0, 0))]),
        out_specs=pl.BlockSpec((B, 40), lambda i: (0, 0)),
        out_shape=jax.ShapeDtypeStruct((B, 40), jnp.float32),
    )(*gs, wc, bc2)


def kernel(pos, batch, W1a, b1a, W1b, b1b, W2a, b2a, W2b, b2b, Wc, bc):
    batch = batch.astype(jnp.int32)
    pos = pos.astype(jnp.float32)

    pos_pad = jnp.pad(pos, ((0, 0), (0, PF - 3)))
    posT = pos.T
    batch2d = batch.reshape(N, 1)
    batchT = batch.reshape(1, N)

    starts = jnp.searchsorted(batch, jnp.arange(B, dtype=jnp.int32),
                              side="left")
    ends = jnp.searchsorted(batch, jnp.arange(B, dtype=jnp.int32),
                            side="right")
    brows = batch.reshape(NR, RB)
    jlo = (starts[brows[:, 0]] // CB).astype(jnp.int32)
    jhi = ((ends[brows[:, -1]] - 1) // CB).astype(jnp.int32)
    sp = jnp.stack([jlo, jhi]).astype(jnp.int32)

    nrp = NR // NP
    flats = []
    for p in range(NP):
        idx_p = _knn_idx(sp[:, p * nrp:(p + 1) * nrp], pos, posT,
                         batch2d, batchT, p)
        flats.append(idx_p.reshape(-1))

    w1_top = jnp.pad(W1a[0:3], ((0, PF - 3), (0, 0)))
    w1_bot = jnp.pad(W1a[3:6], ((0, PF - 3), (0, 0)))
    w1_sum = w1_top + w1_bot
    w2ah = jnp.pad(W2a[0:32], ((0, PF - 32), (0, 0)))
    w2ap = jnp.pad(W2a[32:35], ((0, PF - 3), (0, 0)))
    b1a2 = b1a.reshape(1, 32)
    b1b2 = b1b.reshape(1, 32)
    b2a2 = b2a.reshape(1, 32)
    b2b2 = b2b.reshape(1, 32)
    bc2 = bc.reshape(1, 40)

    hn = N // NP

    pjs = [_sc_gather(pos_pad, f, PF).reshape(K, hn, PF) for f in flats]

    h1s = [_run_layer1(pjs[p], pos_pad, w1_sum, w1_bot, b1a2, W1b, b1b2, p)
           for p in range(NP)]
    h1p = jnp.pad(jnp.concatenate(h1s, axis=0), ((0, 0), (0, PF - 32)))

    hjs = [_sc_gather(h1p, f, PF).reshape(K, hn, PF) for f in flats]

    gs = [_run_layer2(hjs[p], pjs[p], pos_pad, batch2d, w2ah, w2ap, b2a2,
                      W2b, b2b2, p) for p in range(NP)]

    return _combine(gs, Wc, bc2)
